# Initial kernel scaffold; baseline (speedup 1.0000x reference)
import jax
import jax.numpy as jnp
from jax.experimental import pallas as pl


def kernel(x, edge_index, W_rel1, b_rel1, W_root1, W_rel2, b_rel2, W_root2):
    raise NotImplementedError("write your pallas kernel here")


# speedup vs baseline: 22.4823x; 22.4823x over previous
import functools

import jax
import jax.numpy as jnp
from jax import lax
from jax.experimental import pallas as pl
from jax.experimental.pallas import tpu as pltpu
from jax.experimental.pallas import tpu_sc as plsc

_N = 50000
_E = 800000
_HID = 128

_NC = 2
_NS = 16
_NT = _NC * _NS

_NPAD = 50176
_SLICE = _NPAD // _NS
_EPT = 25088
_EPAD = _NT * _EPT
_NCHUNK = 2
_CH = _EPT // _NCHUNK


def _segsum_body(table_hbm, src_hbm, dst_hbm, out_hbm,
                 table_v, src_v, dst_v, vals_v, acc_sh):
    cid = lax.axis_index("c")
    sid = lax.axis_index("s")
    wid = cid * _NS + sid

    pltpu.sync_copy(table_hbm, table_v)

    def _zero_body(i, carry):
        vals_v[pl.ds(i * 16, 16)] = jnp.zeros((16,), jnp.float32)
        return carry

    lax.fori_loop(0, _SLICE // 16, _zero_body, 0)
    pltpu.sync_copy(vals_v.at[pl.ds(0, _SLICE)],
                    acc_sh.at[pl.ds(sid * _SLICE, _SLICE)])
    plsc.subcore_barrier()

    for chunk in range(_NCHUNK):
        base = wid * _EPT + chunk * _CH
        pltpu.sync_copy(src_hbm.at[pl.ds(base, _CH)], src_v)
        pltpu.sync_copy(dst_hbm.at[pl.ds(base, _CH)], dst_v)

        def _gather_body(j, carry):
            o = pl.multiple_of(j * 16, 16)
            idx = src_v[pl.ds(o, 16)]
            vals_v[pl.ds(o, 16)] = plsc.load_gather(table_v, [idx])
            return carry

        lax.fori_loop(0, _CH // 16, _gather_body, 0)

        pltpu.sync_copy(vals_v, acc_sh.at[dst_v], add=True)

    plsc.subcore_barrier()
    pltpu.sync_copy(acc_sh.at[pl.ds(sid * _SLICE, _SLICE)],
                    vals_v.at[pl.ds(0, _SLICE)])
    pltpu.sync_copy(vals_v.at[pl.ds(0, _SLICE)],
                    out_hbm.at[pl.ds(cid * _NPAD + sid * _SLICE, _SLICE)])


_segsum = pl.kernel(
    _segsum_body,
    out_type=jax.ShapeDtypeStruct((_NC * _NPAD,), jnp.float32),
    mesh=plsc.VectorSubcoreMesh(core_axis_name="c", subcore_axis_name="s"),
    compiler_params=pltpu.CompilerParams(needs_layout_passes=False),
    scratch_types=[
        pltpu.VMEM((_NPAD,), jnp.float32),
        pltpu.VMEM((_CH,), jnp.int32),
        pltpu.VMEM((_CH,), jnp.int32),
        pltpu.VMEM((_CH,), jnp.float32),
        pltpu.VMEM_SHARED((_NPAD,), jnp.float32),
    ],
)


_RB = 512
_GRID = _NPAD // _RB


def _dense_kern(spart_ref, x_ref, wa_ref, wb_ref, wc_ref, w2_ref, wr2_ref,
                t_ref, u_ref):
    s = spart_ref[0] + spart_ref[1]
    h = jnp.maximum(s * wa_ref[...] + wb_ref[...] + x_ref[...] * wc_ref[...],
                    0.0)
    t_ref[...] = jnp.sum(h * w2_ref[...], axis=1, keepdims=True)
    u_ref[...] = jnp.sum(h * wr2_ref[...], axis=1, keepdims=True)


def _dense(s_part, x_col, wa, wb, wc, w2, wr2):
    wspec = pl.BlockSpec((1, _HID), lambda i: (0, 0))
    colspec = pl.BlockSpec((_RB, 1), lambda i: (i, 0))
    return pl.pallas_call(
        _dense_kern,
        grid=(_GRID,),
        in_specs=[pl.BlockSpec((_NC, _RB, 1), lambda i: (0, i, 0)),
                  colspec, wspec, wspec, wspec, wspec, wspec],
        out_specs=[colspec, colspec],
        out_shape=[jax.ShapeDtypeStruct((_NPAD, 1), jnp.float32)] * 2,
    )(s_part, x_col, wa, wb, wc, w2, wr2)


def _final_kern(s2_ref, u_ref, b2_ref, emb_ref, upd_ref):
    e = s2_ref[0] + s2_ref[1] + u_ref[...] + b2_ref[0, 0]
    emb_ref[...] = e
    upd_ref[...] = jnp.maximum(e, 0.0)


def _final(s2_part, u_col, b2):
    colspec = pl.BlockSpec((_RB, 1), lambda i: (i, 0))
    return pl.pallas_call(
        _final_kern,
        grid=(_GRID,),
        in_specs=[pl.BlockSpec((_NC, _RB, 1), lambda i: (0, i, 0)),
                  colspec,
                  pl.BlockSpec((1, 1), lambda i: (0, 0))],
        out_specs=[colspec, colspec],
        out_shape=[jax.ShapeDtypeStruct((_NPAD, 1), jnp.float32)] * 2,
    )(s2_part, u_col, b2)


def kernel(x, edge_index, W_rel1, b_rel1, W_root1, W_rel2, b_rel2, W_root2):
    xf = x[:, 0]
    x_pad = jnp.pad(xf, (0, _NPAD - _N))
    src_pad = jnp.pad(edge_index[0], (0, _EPAD - _E))
    dst_pad = jnp.pad(edge_index[1], (0, _EPAD - _E),
                      constant_values=_NPAD - 1)

    s_part = _segsum(x_pad, src_pad, dst_pad)

    wa = W_rel1.reshape(1, _HID)
    wb = b_rel1.reshape(1, _HID)
    wc = W_root1.reshape(1, _HID)
    w2 = W_rel2.reshape(1, _HID)
    wr2 = W_root2.reshape(1, _HID)
    t_col, u_col = _dense(s_part.reshape(_NC, _NPAD, 1),
                          x_pad.reshape(_NPAD, 1), wa, wb, wc, w2, wr2)

    s2_part = _segsum(t_col.reshape(_NPAD), src_pad, dst_pad)

    emb, upd = _final(s2_part.reshape(_NC, _NPAD, 1), u_col,
                      b_rel2.reshape(1, 1))
    return (emb[:_N], upd[:_N])
